# baseline (speedup 1.0000x reference)
import functools

import jax
import jax.numpy as jnp
from jax import lax
from jax.experimental import pallas as pl
from jax.experimental.pallas import tpu as pltpu
from jax.experimental.pallas import tpu_sc as plsc

_NC = 2
_NS = 16
_NW = _NC * _NS
_CHUNK_ROWS = 16
_LANES = 16


def _make_sc_kernel(rows, D, n_table):
    rows_per_w = rows // _NW
    n_chunks = rows_per_w // _CHUNK_ROWS
    chunk_elems = _CHUNK_ROWS * D
    mesh = plsc.VectorSubcoreMesh(core_axis_name="c", subcore_axis_name="s")

    @functools.partial(
        pl.kernel,
        mesh=mesh,
        out_type=jax.ShapeDtypeStruct((rows * D,), jnp.float32),
        scratch_types=[
            pltpu.VMEM((16,), jnp.int32),
            pltpu.VMEM((D,), jnp.float32),
            pltpu.VMEM((chunk_elems,), jnp.float32),
        ],
    )
    def sc_kernel(idx_hbm, x_hbm, emb_hbm, out_hbm, idx_v, emb_v, buf_v):
        wid = lax.axis_index("s") * _NC + lax.axis_index("c")
        base = wid * rows_per_w * D
        pltpu.sync_copy(idx_hbm, idx_v)
        row_idx = idx_v[pl.ds(0, _LANES)][0]
        pltpu.sync_copy(emb_hbm.at[pl.ds(row_idx * D, D)], emb_v)

        def chunk_body(g):
            off = base + g * chunk_elems
            pltpu.sync_copy(x_hbm.at[pl.ds(off, chunk_elems)], buf_v)

            def row_body(r):
                rbase = r * D
                for j in range(D // _LANES):
                    sl = pl.ds(rbase + j * _LANES, _LANES)
                    buf_v[sl] = buf_v[sl] + emb_v[pl.ds(j * _LANES, _LANES)]

            pl.loop(0, _CHUNK_ROWS)(row_body)
            pltpu.sync_copy(buf_v, out_hbm.at[pl.ds(off, chunk_elems)])

        pl.loop(0, n_chunks)(chunk_body)

    return sc_kernel


def kernel(x, layer_idx, step_embeddings):
    B, S, D = x.shape
    rows = B * S
    n_table = step_embeddings.shape[0]
    x_flat = x.reshape(rows * D)
    emb_flat = step_embeddings.reshape(n_table * D)
    idx = jnp.asarray(layer_idx, dtype=jnp.int32).reshape(1)
    idx16 = jnp.pad(idx, (0, 15))
    sc = _make_sc_kernel(rows, D, n_table)
    out = sc(idx16, x_flat, emb_flat)
    return out.reshape(B, S, D)


# speedup vs baseline: 2.2878x; 2.2878x over previous
import functools

import jax
import jax.numpy as jnp
from jax import lax
from jax.experimental import pallas as pl
from jax.experimental.pallas import tpu as pltpu
from jax.experimental.pallas import tpu_sc as plsc

_NC = 2
_NS = 16
_NW = _NC * _NS
_CHUNK_ROWS = 16
_LANES = 16
_BLOCK_ROWS = 1024
_SC_ROWS = 1024


def _tc_add_body(idx_ref, x_ref, emb_ref, o_ref):
    row = emb_ref[idx_ref[0]]
    o_ref[...] = x_ref[...] + row


def _tc_call(idx, x2, step_embeddings, tc_rows):
    rows, D = x2.shape
    n_table = step_embeddings.shape[0]
    block = min(_BLOCK_ROWS, tc_rows)
    grid = tc_rows // block
    return pl.pallas_call(
        _tc_add_body,
        grid=(grid,),
        in_specs=[
            pl.BlockSpec(memory_space=pltpu.SMEM),
            pl.BlockSpec((block, D), lambda i: (i, 0)),
            pl.BlockSpec((n_table, D), lambda i: (0, 0)),
        ],
        out_specs=pl.BlockSpec((block, D), lambda i: (i, 0)),
        out_shape=jax.ShapeDtypeStruct((tc_rows, D), x2.dtype),
        compiler_params=pltpu.CompilerParams(
            dimension_semantics=("parallel",),
        ),
    )(idx, x2, step_embeddings)


def _make_sc_kernel(total_rows, sc_rows, D):
    base_row = total_rows - sc_rows
    rows_per_w = sc_rows // _NW
    n_chunks = rows_per_w // _CHUNK_ROWS
    chunk_elems = _CHUNK_ROWS * D
    mesh = plsc.VectorSubcoreMesh(core_axis_name="c", subcore_axis_name="s")

    @functools.partial(
        pl.kernel,
        mesh=mesh,
        out_type=jax.ShapeDtypeStruct((sc_rows * D,), jnp.float32),
        scratch_types=[
            pltpu.VMEM((16,), jnp.int32),
            pltpu.VMEM((D,), jnp.float32),
            pltpu.VMEM((chunk_elems,), jnp.float32),
        ],
    )
    def sc_kernel(idx_hbm, x_hbm, emb_hbm, out_hbm, idx_v, emb_v, buf_v):
        wid = lax.axis_index("s") * _NC + lax.axis_index("c")
        in_base = (base_row + wid * rows_per_w) * D
        out_base = wid * rows_per_w * D
        pltpu.sync_copy(idx_hbm, idx_v)
        row_idx = idx_v[pl.ds(0, _LANES)][0]
        pltpu.sync_copy(emb_hbm.at[pl.ds(row_idx * D, D)], emb_v)

        def chunk_body(g):
            pltpu.sync_copy(
                x_hbm.at[pl.ds(in_base + g * chunk_elems, chunk_elems)], buf_v
            )

            def row_body(r):
                rbase = r * D
                for j in range(D // _LANES):
                    sl = pl.ds(rbase + j * _LANES, _LANES)
                    buf_v[sl] = buf_v[sl] + emb_v[pl.ds(j * _LANES, _LANES)]

            pl.loop(0, _CHUNK_ROWS)(row_body)
            pltpu.sync_copy(
                buf_v, out_hbm.at[pl.ds(out_base + g * chunk_elems, chunk_elems)]
            )

        pl.loop(0, n_chunks)(chunk_body)

    return sc_kernel


def kernel(x, layer_idx, step_embeddings):
    B, S, D = x.shape
    rows = B * S
    n_table = step_embeddings.shape[0]
    x2 = x.reshape(rows, D)
    x_flat = x.reshape(rows * D)
    emb_flat = step_embeddings.reshape(n_table * D)
    idx = jnp.asarray(layer_idx, dtype=jnp.int32).reshape(1)
    idx16 = jnp.pad(idx, (0, 15))

    tc_rows = rows - _SC_ROWS
    out_tc = _tc_call(idx, x2, step_embeddings, tc_rows)
    sc = _make_sc_kernel(rows, _SC_ROWS, D)
    out_sc = sc(idx16, x_flat, emb_flat).reshape(_SC_ROWS, D)
    out = jnp.concatenate([out_tc, out_sc], axis=0)
    return out.reshape(B, S, D)


# speedup vs baseline: 7.8658x; 3.4382x over previous
import jax
import jax.numpy as jnp
from jax.experimental import pallas as pl
from jax.experimental.pallas import tpu as pltpu

_BLOCK_ROWS = 2048


def _add_body(idx_ref, x_ref, emb_ref, o_ref):
    row = emb_ref[idx_ref[0]]
    o_ref[...] = x_ref[...] + row


def kernel(x, layer_idx, step_embeddings):
    B, S, D = x.shape
    rows = B * S
    x2 = x.reshape(rows, D)
    n_table = step_embeddings.shape[0]
    block = min(_BLOCK_ROWS, rows)
    grid = rows // block
    idx = jnp.asarray(layer_idx, dtype=jnp.int32).reshape(1)
    out = pl.pallas_call(
        _add_body,
        grid=(grid, 2),
        in_specs=[
            pl.BlockSpec(memory_space=pltpu.SMEM),
            pl.BlockSpec((block, D // 2), lambda i, j: (i, j)),
            pl.BlockSpec((n_table, D // 2), lambda i, j: (0, j)),
        ],
        out_specs=pl.BlockSpec((block, D // 2), lambda i, j: (i, j)),
        out_shape=jax.ShapeDtypeStruct((rows, D), x.dtype),
        compiler_params=pltpu.CompilerParams(
            dimension_semantics=("parallel", "parallel"),
        ),
    )(idx, x2, step_embeddings)
    return out.reshape(B, S, D)


# speedup vs baseline: 7.9390x; 1.0093x over previous
import jax
import jax.numpy as jnp
from jax.experimental import pallas as pl
from jax.experimental.pallas import tpu as pltpu

_BLOCK_ROWS = 1024


def _add_body(idx_ref, x_ref, emb_ref, o_ref):
    row = emb_ref[idx_ref[0]]
    o_ref[...] = x_ref[...] + row


def kernel(x, layer_idx, step_embeddings):
    B, S, D = x.shape
    rows = B * S
    x2 = x.reshape(rows, D)
    n_table = step_embeddings.shape[0]
    block = min(_BLOCK_ROWS, rows)
    grid = rows // block
    idx = jnp.asarray(layer_idx, dtype=jnp.int32).reshape(1)
    out = pl.pallas_call(
        _add_body,
        grid=(grid,),
        in_specs=[
            pl.BlockSpec(memory_space=pltpu.SMEM),
            pl.BlockSpec((block, D), lambda i: (i, 0)),
            pl.BlockSpec((n_table, D), lambda i: (0, 0)),
        ],
        out_specs=pl.BlockSpec((block, D), lambda i: (i, 0)),
        out_shape=jax.ShapeDtypeStruct((rows, D), x.dtype),
        compiler_params=pltpu.CompilerParams(
            dimension_semantics=("parallel",),
        ),
    )(idx, x2, step_embeddings)
    return out.reshape(B, S, D)
